# baseline (speedup 1.0000x reference)
import functools

import jax
import jax.numpy as jnp
from jax import lax
from jax.experimental import pallas as pl
from jax.experimental.pallas import tpu as pltpu
from jax.experimental.pallas import tpu_sc as plsc

B = 8
L = 4096
MIN_DIST = 32
NLANES = 16
NBLOCKS = 32
BLOCK = L // NBLOCKS
BLOCK_SHIFT = 7
CHUNKS_PER_BLOCK = BLOCK // NLANES
NEG = float("-inf")
BIGI = 1 << 30


def _nms_body(x_hbm, out_hbm, x_v, vals_v, out_v):
    w = lax.axis_index("s") * 2 + lax.axis_index("c")

    @pl.when(w < B)
    def _():
        b = w
        pltpu.sync_copy(x_hbm.at[b], x_v)

        lane = lax.broadcasted_iota(jnp.int32, (NLANES,), 0)
        zeros = jnp.zeros((NLANES,), jnp.float32)
        ninf = jnp.full((NLANES,), NEG, jnp.float32)
        bigi = jnp.full((NLANES,), BIGI, jnp.int32)

        def bupdate(jj, bm, b0, b1):
            sel = lane == (jj & (NLANES - 1))
            lo_half = jj < NLANES
            b0 = jnp.where(sel & lo_half, bm, b0)
            b1 = jnp.where(sel & (~lo_half), bm, b1)
            return b0, b1

        def block_pass(j, carry):
            b0, b1 = carry

            acc = ninf
            for c in range(CHUNKS_PER_BLOCK):
                idx = j * BLOCK + c * NLANES + lane
                xc = plsc.load_gather(x_v, [idx])
                xm = plsc.load_gather(x_v, [jnp.maximum(idx - 1, 0)])
                xr = plsc.load_gather(x_v, [jnp.minimum(idx + 1, L - 1)])
                dl = xc - xm
                dr = xr - xc
                pos = xc > 0.0
                peak = pos & (dr <= 0.0) & (dl > 0.0)
                valley = (~pos) & (dr > 0.0) & (dl <= 0.0)
                v = jnp.where(peak | valley, jnp.abs(xc), NEG)
                plsc.store_scatter(vals_v, [idx], v)
                plsc.store_scatter(out_v, [idx], zeros)
                acc = jnp.maximum(acc, v)
            return bupdate(jnp.full((NLANES,), j, jnp.int32), jnp.max(acc), b0, b1)

        b0, b1 = lax.fori_loop(0, NBLOCKS, block_pass, (ninf, ninf))

        def greedy_cond(carry):
            m = carry[0]
            return m > NEG

        def greedy_body(carry):
            m, b0, b1 = carry
            f0 = plsc.all_reduce_ffs(b0 == m)
            f1 = plsc.all_reduce_ffs(b1 == m)
            j = jnp.where(f0 < NLANES, f0, f1 + NLANES)
            base0 = j << BLOCK_SHIFT

            cands = []
            for c in range(CHUNKS_PER_BLOCK):
                idx = base0 + c * NLANES + lane
                v = plsc.load_gather(vals_v, [idx])
                f = plsc.all_reduce_ffs(v == m)
                cands.append(jnp.where(f < NLANES, base0 + c * NLANES + f, bigi))
            while len(cands) > 1:
                cands = [
                    jnp.minimum(a, b) for a, b in zip(cands[::2], cands[1::2])
                ]
            p = cands[0]

            xp = plsc.load_gather(x_v, [p])
            plsc.store_scatter(out_v, [p], xp, mask=lane == 0)

            lo = jnp.maximum(p - MIN_DIST, 0)
            hi = jnp.minimum(p + MIN_DIST, L - 1)
            for k in range((2 * MIN_DIST) // NLANES + 1):
                sidx = lo + k * NLANES + lane
                plsc.store_scatter(
                    vals_v, [jnp.minimum(sidx, L - 1)], ninf, mask=sidx <= hi
                )

            ja = p >> BLOCK_SHIFT
            jb = (lo >> BLOCK_SHIFT) + (hi >> BLOCK_SHIFT) - ja
            basea = ja << BLOCK_SHIFT
            baseb = jb << BLOCK_SHIFT

            vas, vbs = [], []
            for c in range(CHUNKS_PER_BLOCK):
                off = c * NLANES + lane
                vas.append(plsc.load_gather(vals_v, [basea + off]))
                vbs.append(plsc.load_gather(vals_v, [baseb + off]))
            while len(vas) > 1:
                vas = [jnp.maximum(a, b) for a, b in zip(vas[::2], vas[1::2])]
                vbs = [jnp.maximum(a, b) for a, b in zip(vbs[::2], vbs[1::2])]
            acca, accb = vas[0], vbs[0]
            b0, b1 = bupdate(ja, jnp.max(acca), b0, b1)
            b0, b1 = bupdate(jb, jnp.max(accb), b0, b1)
            return jnp.max(jnp.maximum(b0, b1)), b0, b1

        m0 = jnp.max(jnp.maximum(b0, b1))
        lax.while_loop(greedy_cond, greedy_body, (m0, b0, b1))

        pltpu.sync_copy(out_v, out_hbm.at[b])


@jax.jit
def _nms(x):
    run = pl.kernel(
        _nms_body,
        out_type=jax.ShapeDtypeStruct((B, L), jnp.float32),
        mesh=plsc.VectorSubcoreMesh(core_axis_name="c", subcore_axis_name="s"),
        compiler_params=pltpu.CompilerParams(needs_layout_passes=False),
        scratch_types=[
            pltpu.VMEM((L,), jnp.float32),
            pltpu.VMEM((L,), jnp.float32),
            pltpu.VMEM((L,), jnp.float32),
        ],
    )
    return run(x)


def kernel(input_):
    return _nms(input_.reshape(B, L)).reshape(B, 1, L)


# speedup vs baseline: 1.0151x; 1.0151x over previous
import functools

import jax
import jax.numpy as jnp
from jax import lax
from jax.experimental import pallas as pl
from jax.experimental.pallas import tpu as pltpu
from jax.experimental.pallas import tpu_sc as plsc

B = 8
L = 4096
MIN_DIST = 32
NLANES = 16
NBLOCKS = 32
BLOCK = L // NBLOCKS
BLOCK_SHIFT = 7
CHUNKS_PER_BLOCK = BLOCK // NLANES
NEG = float("-inf")
BIGI = 1 << 30


def _nms_body(x_hbm, out_hbm, x_v, vals_v, out_v):
    w = lax.axis_index("s") * 2 + lax.axis_index("c")

    @pl.when(w < B)
    def _():
        b = w
        pltpu.sync_copy(x_hbm.at[b], x_v)

        lane = lax.broadcasted_iota(jnp.int32, (NLANES,), 0)
        zeros = jnp.zeros((NLANES,), jnp.float32)
        ninf = jnp.full((NLANES,), NEG, jnp.float32)
        bigi = jnp.full((NLANES,), BIGI, jnp.int32)

        def bupdate(jj, bm, b0, b1):
            sel = lane == (jj & (NLANES - 1))
            lo_half = jj < NLANES
            b0 = jnp.where(sel & lo_half, bm, b0)
            b1 = jnp.where(sel & (~lo_half), bm, b1)
            return b0, b1

        def block_pass(j, carry):
            b0, b1 = carry

            acc = ninf
            for c in range(CHUNKS_PER_BLOCK):
                idx = j * BLOCK + c * NLANES + lane
                xc = plsc.load_gather(x_v, [idx])
                xm = plsc.load_gather(x_v, [jnp.maximum(idx - 1, 0)])
                xr = plsc.load_gather(x_v, [jnp.minimum(idx + 1, L - 1)])
                dl = xc - xm
                dr = xr - xc
                pos = xc > 0.0
                peak = pos & (dr <= 0.0) & (dl > 0.0)
                valley = (~pos) & (dr > 0.0) & (dl <= 0.0)
                v = jnp.where(peak | valley, jnp.abs(xc), NEG)
                plsc.store_scatter(vals_v, [idx], v)
                plsc.store_scatter(out_v, [idx], zeros)
                acc = jnp.maximum(acc, v)
            return bupdate(jnp.full((NLANES,), j, jnp.int32), jnp.max(acc), b0, b1)

        b0, b1 = lax.fori_loop(0, NBLOCKS, block_pass, (ninf, ninf))

        def greedy_cond(carry):
            m = carry[0]
            return m > NEG

        def greedy_body(carry):
            m, b0, b1 = carry
            f0 = plsc.all_reduce_ffs(b0 == m)
            f1 = plsc.all_reduce_ffs(b1 == m)
            j = jnp.where(f0 < NLANES, f0, f1 + NLANES)
            base0 = j << BLOCK_SHIFT

            cands = []
            for c in range(CHUNKS_PER_BLOCK):
                idx = base0 + c * NLANES + lane
                v = plsc.load_gather(vals_v, [idx])
                f = plsc.all_reduce_ffs(v == m)
                cands.append(jnp.where(f < NLANES, base0 + c * NLANES + f, bigi))
            while len(cands) > 1:
                cands = [
                    jnp.minimum(a, b) for a, b in zip(cands[::2], cands[1::2])
                ]
            p = cands[0]

            xp = plsc.load_gather(x_v, [p])
            plsc.store_scatter(out_v, [p], xp, mask=lane == 0)

            lo = jnp.maximum(p - MIN_DIST, 0)
            hi = jnp.minimum(p + MIN_DIST, L - 1)
            for k in range((2 * MIN_DIST) // NLANES + 1):
                sidx = lo + k * NLANES + lane
                plsc.store_scatter(
                    vals_v, [jnp.minimum(sidx, L - 1)], ninf, mask=sidx <= hi
                )

            ja = p >> BLOCK_SHIFT
            jb = (lo >> BLOCK_SHIFT) + (hi >> BLOCK_SHIFT) - ja
            basea = ja << BLOCK_SHIFT
            baseb = jb << BLOCK_SHIFT

            exa = lane == (ja & (NLANES - 1))
            exb = lane == (jb & (NLANES - 1))
            b0m = jnp.where((exa & (ja < NLANES)) | (exb & (jb < NLANES)), NEG, b0)
            b1m = jnp.where((exa & (ja >= NLANES)) | (exb & (jb >= NLANES)), NEG, b1)
            mpre = jnp.max(jnp.maximum(b0m, b1m))

            vas, vbs = [], []
            for c in range(CHUNKS_PER_BLOCK):
                off = c * NLANES + lane
                vas.append(plsc.load_gather(vals_v, [basea + off]))
                vbs.append(plsc.load_gather(vals_v, [baseb + off]))
            while len(vas) > 1:
                vas = [jnp.maximum(a, b) for a, b in zip(vas[::2], vas[1::2])]
                vbs = [jnp.maximum(a, b) for a, b in zip(vbs[::2], vbs[1::2])]
            bma = jnp.max(vas[0])
            bmb = jnp.max(vbs[0])
            b0, b1 = bupdate(ja, bma, b0, b1)
            b0, b1 = bupdate(jb, bmb, b0, b1)
            return jnp.maximum(jnp.maximum(mpre, bma), bmb), b0, b1

        m0 = jnp.max(jnp.maximum(b0, b1))
        lax.while_loop(greedy_cond, greedy_body, (m0, b0, b1))

        pltpu.sync_copy(out_v, out_hbm.at[b])


@jax.jit
def _nms(x):
    run = pl.kernel(
        _nms_body,
        out_type=jax.ShapeDtypeStruct((B, L), jnp.float32),
        mesh=plsc.VectorSubcoreMesh(core_axis_name="c", subcore_axis_name="s"),
        compiler_params=pltpu.CompilerParams(needs_layout_passes=False),
        scratch_types=[
            pltpu.VMEM((L,), jnp.float32),
            pltpu.VMEM((L,), jnp.float32),
            pltpu.VMEM((L,), jnp.float32),
        ],
    )
    return run(x)


def kernel(input_):
    return _nms(input_.reshape(B, L)).reshape(B, 1, L)
